# Initial kernel scaffold; baseline (speedup 1.0000x reference)
import jax
import jax.numpy as jnp
from jax.experimental import pallas as pl


def kernel(x_variant, x_gene, x_complex, x_phenotype, ei_vg, ei_gv, ei_vp, ei_pv, ei_gc, ei_cg, Wp_variant, bp_variant, Wp_gene, bp_gene, Wp_complex, bp_complex, Wp_phenotype, bp_phenotype, Wl1_vg, bl1_vg, Wr1_vg, Wl2_vg, bl2_vg, Wr2_vg, Wl1_gv, bl1_gv, Wr1_gv, Wl2_gv, bl2_gv, Wr2_gv, Wl1_vp, bl1_vp, Wr1_vp, Wl2_vp, bl2_vp, Wr2_vp, Wl1_pv, bl1_pv, Wr1_pv, Wl2_pv, bl2_pv, Wr2_pv, Wl1_gc, bl1_gc, Wr1_gc, Wl2_gc, bl2_gc, Wr2_gc, Wl1_cg, bl1_cg, Wr1_cg, Wl2_cg, bl2_cg, Wr2_cg):
    raise NotImplementedError("write your pallas kernel here")


# speedup vs baseline: 2.3778x; 2.3778x over previous
import functools

import jax
import jax.numpy as jnp
from jax import lax
from jax.experimental import pallas as pl
from jax.experimental.pallas import tpu as pltpu
from jax.experimental.pallas import tpu_sc as plsc

_NV, _NG, _NC, _NP = 100000, 20000, 2000, 8000
_H, _O = 64, 32

_NTILES = 16
_G = 8


def _rup(x, m):
    return (x + m - 1) // m * m


_PAD = {"variant": _rup(_NV + 1, 128), "gene": _rup(_NG + 1, 128),
        "complex": _rup(_NC + 1, 128), "phenotype": _rup(_NP + 1, 128)}


def _sc_mesh():
    return plsc.VectorSubcoreMesh(core_axis_name="c", subcore_axis_name="s",
                                  num_cores=1)


_SPMEM_WORDS = 2_000_000


def _pick_g(n_acc, dc, rpt):
    for g in (8, 4, 2, 1):
        b = g * 128
        zr = min(256 if dc <= 32 else 128, rpt)
        if n_acc * dc + 16 * (b * dc + zr * dc + 2 * b) <= _SPMEM_WORDS:
            return g, zr
    raise ValueError("accumulator too large for Spmem")


@functools.lru_cache(maxsize=None)
def _segsum_call(n_acc, dc, n_rows):
    rpt = n_acc // 16
    g, zr = _pick_g(n_acc, dc, rpt)
    b = g * 128
    n_grp = n_rows // (16 * g)

    def body(tab, srci, dsti, out, src_v, dst_v, rows_v, zbuf, acc, gsem, ssem):
        sid = lax.axis_index("s")
        tid = sid

        z16 = jnp.zeros((16,), jnp.float32)

        def zrow(i, c):
            for jj in range(dc // 16):
                zbuf[i, pl.ds(jj * 16, 16)] = z16
            return c

        lax.fori_loop(0, zr, zrow, 0)

        base = sid * rpt
        nfull, rem = rpt // zr, rpt % zr

        def zacc(i, c):
            pltpu.sync_copy(zbuf, acc.at[pl.ds(base + i * zr, zr)])
            return c

        lax.fori_loop(0, nfull, zacc, 0)
        if rem:
            pltpu.sync_copy(zbuf.at[pl.ds(0, rem)],
                            acc.at[pl.ds(base + nfull * zr, rem)])
        plsc.subcore_barrier()

        def grp(gi, c):
            row0 = tid * (n_grp * g) + gi * g
            pltpu.sync_copy(srci.at[pl.ds(row0, g)], src_v)
            pltpu.sync_copy(dsti.at[pl.ds(row0, g)], dst_v)
            cps = [pltpu.async_copy(tab.at[src_v.at[j]],
                                    rows_v.at[pl.ds(j * 128, 128)], gsem)
                   for j in range(g)]
            for cp in cps:
                cp.wait()
            cps = [pltpu.async_copy(rows_v.at[pl.ds(j * 128, 128)],
                                    acc.at[dst_v.at[j]], ssem, add=True)
                   for j in range(g)]
            for cp in cps:
                cp.wait()
            return c

        lax.fori_loop(0, n_grp, grp, 0)
        plsc.subcore_barrier()

        nfl, refl = rpt // zr, rpt % zr

        def fl(i, c):
            pltpu.sync_copy(acc.at[pl.ds(base + i * zr, zr)], zbuf)
            pltpu.sync_copy(zbuf, out.at[pl.ds(base + i * zr, zr)])
            return c

        lax.fori_loop(0, nfl, fl, 0)
        if refl:
            pltpu.sync_copy(acc.at[pl.ds(base + nfl * zr, refl)],
                            zbuf.at[pl.ds(0, refl)])
            pltpu.sync_copy(zbuf.at[pl.ds(0, refl)],
                            out.at[pl.ds(base + nfl * zr, refl)])

    return pl.kernel(
        body,
        out_type=jax.ShapeDtypeStruct((n_acc, dc), jnp.float32),
        compiler_params=pltpu.CompilerParams(use_tc_tiling_on_sc=False),
        mesh=_sc_mesh(),
        scratch_types=[
            pltpu.VMEM((g, 128), jnp.int32),
            pltpu.VMEM((g, 128), jnp.int32),
            pltpu.VMEM((b, dc), jnp.float32),
            pltpu.VMEM((zr, dc), jnp.float32),
            pltpu.VMEM_SHARED((n_acc, dc), jnp.float32),
            pltpu.SemaphoreType.DMA,
            pltpu.SemaphoreType.DMA,
        ],
    )


def _pad_edges(ei, n_dst):
    e = ei.shape[1]
    unit = _NTILES * _G * 128
    e_pad = _rup(e, unit)
    src = jnp.concatenate([ei[0], jnp.zeros((e_pad - e,), ei.dtype)])
    dst = jnp.concatenate([ei[1], jnp.full((e_pad - e,), n_dst, ei.dtype)])
    return (src.astype(jnp.int32).reshape(-1, 128),
            dst.astype(jnp.int32).reshape(-1, 128))


def _segsum(tab, srci, dsti, n_acc):
    dc = tab.shape[1]
    return _segsum_call(n_acc, dc, srci.shape[0])(tab, srci, dsti)


def _segcnt(dsti, n_acc):
    ones_tab = jnp.ones((n_acc, 16), jnp.float32)
    return _segsum_call(n_acc, 16, dsti.shape[0])(ones_tab, dsti, dsti)[:, :1]


_BR = 512


def _dot_t(a, w):
    return lax.dot_general(a, w, (((1,), (1,)), ((), ())),
                           preferred_element_type=jnp.float32,
                           precision=lax.Precision.HIGHEST)


@functools.lru_cache(maxsize=None)
def _proj_call(n, k_in, m):
    def body(x_ref, w_ref, b_ref, o_ref):
        o_ref[...] = _dot_t(x_ref[...], w_ref[...]) + b_ref[...]

    grid = pl.cdiv(n, _BR)
    return pl.pallas_call(
        body,
        grid=(grid,),
        in_specs=[
            pl.BlockSpec((_BR, k_in), lambda i: (i, 0)),
            pl.BlockSpec((m, k_in), lambda i: (0, 0)),
            pl.BlockSpec((1, m), lambda i: (0, 0)),
        ],
        out_specs=pl.BlockSpec((_BR, m), lambda i: (i, 0)),
        out_shape=jax.ShapeDtypeStruct((n, m), jnp.float32),
    )


@functools.lru_cache(maxsize=None)
def _dual_proj_call(n, k_in, m):
    def body(x_ref, w1_ref, w2_ref, o1_ref, o2_ref):
        x = x_ref[...]
        o1_ref[...] = _dot_t(x, w1_ref[...])
        o2_ref[...] = _dot_t(x, w2_ref[...])

    grid = pl.cdiv(n, _BR)
    return pl.pallas_call(
        body,
        grid=(grid,),
        in_specs=[
            pl.BlockSpec((_BR, k_in), lambda i: (i, 0)),
            pl.BlockSpec((m, k_in), lambda i: (0, 0)),
            pl.BlockSpec((m, k_in), lambda i: (0, 0)),
        ],
        out_specs=[pl.BlockSpec((_BR, m), lambda i: (i, 0)),
                   pl.BlockSpec((_BR, m), lambda i: (i, 0))],
        out_shape=[jax.ShapeDtypeStruct((n, m), jnp.float32),
                   jax.ShapeDtypeStruct((n, m), jnp.float32)],
    )


@functools.lru_cache(maxsize=None)
def _combine_call(n, d_out, n_chunks_a, n_chunks_b, mm_a, mm_b, relu, d_in):
    nrel = 1 if n_chunks_b == 0 else 2
    dc_a = d_in // n_chunks_a
    dc_b = d_in // n_chunks_b if n_chunks_b else 0

    def _mean(chunk_refs, cnt_ref):
        c = cnt_ref[...]
        inv = jnp.where(c > 0, 1.0 / jnp.maximum(c, 1.0), 0.0)
        parts = [r[...] for r in chunk_refs]
        s = parts[0] if len(parts) == 1 else jnp.concatenate(parts, axis=1)
        return s * inv

    def body(*refs):
        refs = list(refs)
        sa = [refs.pop(0) for _ in range(n_chunks_a)]
        ca = refs.pop(0)
        wla = refs.pop(0) if mm_a else None
        sb = [refs.pop(0) for _ in range(n_chunks_b)]
        cb = refs.pop(0) if n_chunks_b else None
        wlb = refs.pop(0) if mm_b else None
        h_ref, wra_ref, b_ref = refs[0], refs[1], refs[2]
        wrb_ref = refs[3] if nrel == 2 else None
        o_ref = refs[-1]

        ma = _mean(sa, ca)
        acc = _dot_t(ma, wla[...]) if mm_a else ma
        if n_chunks_b:
            mb = _mean(sb, cb)
            acc = acc + (_dot_t(mb, wlb[...]) if mm_b else mb)
        wr = wra_ref[...]
        if nrel == 2:
            wr = wr + wrb_ref[...]
        acc = acc + _dot_t(h_ref[...], wr) + b_ref[...]
        acc = acc * (1.0 / nrel)
        if relu:
            acc = jnp.maximum(acc, 0.0)
        o_ref[...] = acc

    grid = pl.cdiv(n, _BR)
    in_specs = []
    in_specs += [pl.BlockSpec((_BR, dc_a), lambda i: (i, 0))
                 for _ in range(n_chunks_a)]
    in_specs.append(pl.BlockSpec((_BR, 1), lambda i: (i, 0)))
    if mm_a:
        in_specs.append(pl.BlockSpec((d_out, d_in), lambda i: (0, 0)))
    in_specs += [pl.BlockSpec((_BR, dc_b), lambda i: (i, 0))
                 for _ in range(n_chunks_b)]
    if n_chunks_b:
        in_specs.append(pl.BlockSpec((_BR, 1), lambda i: (i, 0)))
        if mm_b:
            in_specs.append(pl.BlockSpec((d_out, d_in), lambda i: (0, 0)))
    in_specs.append(pl.BlockSpec((_BR, _H), lambda i: (i, 0)))
    in_specs.append(pl.BlockSpec((d_out, _H), lambda i: (0, 0)))
    in_specs.append(pl.BlockSpec((1, d_out), lambda i: (0, 0)))
    if nrel == 2:
        in_specs.append(pl.BlockSpec((d_out, _H), lambda i: (0, 0)))

    return pl.pallas_call(
        body,
        grid=(grid,),
        in_specs=in_specs,
        out_specs=pl.BlockSpec((_BR, d_out), lambda i: (i, 0)),
        out_shape=jax.ShapeDtypeStruct((n, d_out), jnp.float32),
    )


def _pad_rows(x, n_pad):
    return jnp.pad(x, ((0, n_pad - x.shape[0]), (0, 0)))


def kernel(x_variant, x_gene, x_complex, x_phenotype, ei_vg, ei_gv, ei_vp,
           ei_pv, ei_gc, ei_cg, Wp_variant, bp_variant, Wp_gene, bp_gene,
           Wp_complex, bp_complex, Wp_phenotype, bp_phenotype,
           Wl1_vg, bl1_vg, Wr1_vg, Wl2_vg, bl2_vg, Wr2_vg,
           Wl1_gv, bl1_gv, Wr1_gv, Wl2_gv, bl2_gv, Wr2_gv,
           Wl1_vp, bl1_vp, Wr1_vp, Wl2_vp, bl2_vp, Wr2_vp,
           Wl1_pv, bl1_pv, Wr1_pv, Wl2_pv, bl2_pv, Wr2_pv,
           Wl1_gc, bl1_gc, Wr1_gc, Wl2_gc, bl2_gc, Wr2_gc,
           Wl1_cg, bl1_cg, Wr1_cg, Wl2_cg, bl2_cg, Wr2_cg):
    nvp, ngp, ncp, npp = (_PAD["variant"], _PAD["gene"], _PAD["complex"],
                          _PAD["phenotype"])

    e_vg = _pad_edges(ei_vg, _NG)
    e_gv = _pad_edges(ei_gv, _NV)
    e_vp = _pad_edges(ei_vp, _NP)
    e_pv = _pad_edges(ei_pv, _NV)
    e_gc = _pad_edges(ei_gc, _NC)
    e_cg = _pad_edges(ei_cg, _NG)

    c_vg = _segcnt(e_vg[1], ngp)
    c_gv = _segcnt(e_gv[1], nvp)
    c_vp = _segcnt(e_vp[1], npp)
    c_pv = _segcnt(e_pv[1], nvp)
    c_gc = _segcnt(e_gc[1], ncp)
    c_cg = _segcnt(e_cg[1], ngp)

    h0_v = _proj_call(nvp, x_variant.shape[1], _H)(
        _pad_rows(x_variant, nvp), Wp_variant, bp_variant[None])
    h0_g = _proj_call(ngp, x_gene.shape[1], _H)(
        _pad_rows(x_gene, ngp), Wp_gene, bp_gene[None])
    h0_c = _proj_call(ncp, x_complex.shape[1], _H)(
        _pad_rows(x_complex, ncp), Wp_complex, bp_complex[None])
    h0_p = _proj_call(npp, x_phenotype.shape[1], _H)(
        _pad_rows(x_phenotype, npp), Wp_phenotype, bp_phenotype[None])

    zb64 = jnp.zeros((1, _H), jnp.float32)
    y_gv = _proj_call(ngp, _H, _H)(h0_g, Wl1_gv, zb64)
    y_cg = _proj_call(ncp, _H, _H)(h0_c, Wl1_cg, zb64)
    y_pv = _proj_call(npp, _H, _H)(h0_p, Wl1_pv, zb64)

    s_vg = _segsum(h0_v, e_vg[0], e_vg[1], ngp)
    s_vp = _segsum(h0_v, e_vp[0], e_vp[1], npp)
    s_gc = _segsum(h0_g, e_gc[0], e_gc[1], ncp)
    s_cg = _segsum(y_cg, e_cg[0], e_cg[1], ngp)
    s_gv = [_segsum(y_gv[:, c * 16:(c + 1) * 16], e_gv[0], e_gv[1], nvp)
            for c in range(4)]
    s_pv = [_segsum(y_pv[:, c * 16:(c + 1) * 16], e_pv[0], e_pv[1], nvp)
            for c in range(4)]

    b_g = (bl1_vg + bl1_cg)[None]
    h1_g = _combine_call(ngp, _H, 1, 1, True, False, True, _H)(
        s_vg, c_vg, Wl1_vg, s_cg, c_cg, h0_g, Wr1_vg, b_g, Wr1_cg)
    h1_p = _combine_call(npp, _H, 1, 0, True, False, True, _H)(
        s_vp, c_vp, Wl1_vp, h0_p, Wr1_vp, bl1_vp[None])
    h1_c = _combine_call(ncp, _H, 1, 0, True, False, True, _H)(
        s_gc, c_gc, Wl1_gc, h0_c, Wr1_gc, bl1_gc[None])
    b_v = (bl1_gv + bl1_pv)[None]
    h1_v = _combine_call(nvp, _H, 4, 4, False, False, True, _H)(
        *s_gv, c_gv, *s_pv, c_pv, h0_v, Wr1_gv, b_v, Wr1_pv)

    y2_vg, y2_vp = _dual_proj_call(nvp, _H, _O)(h1_v, Wl2_vg, Wl2_vp)
    y2_gv, y2_gc = _dual_proj_call(ngp, _H, _O)(h1_g, Wl2_gv, Wl2_gc)
    y2_pv = _proj_call(npp, _H, _O)(h1_p, Wl2_pv,
                                    jnp.zeros((1, _O), jnp.float32))
    y2_cg = _proj_call(ncp, _H, _O)(h1_c, Wl2_cg,
                                    jnp.zeros((1, _O), jnp.float32))

    s2_vg = _segsum(y2_vg, e_vg[0], e_vg[1], ngp)
    s2_vp = _segsum(y2_vp, e_vp[0], e_vp[1], npp)
    s2_gc = _segsum(y2_gc, e_gc[0], e_gc[1], ncp)
    s2_cg = _segsum(y2_cg, e_cg[0], e_cg[1], ngp)
    s2_gv = [_segsum(y2_gv[:, c * 16:(c + 1) * 16], e_gv[0], e_gv[1], nvp)
             for c in range(2)]
    s2_pv = [_segsum(y2_pv[:, c * 16:(c + 1) * 16], e_pv[0], e_pv[1], nvp)
             for c in range(2)]

    b2_g = (bl2_vg + bl2_cg)[None]
    o_g = _combine_call(ngp, _O, 1, 1, False, False, False, _O)(
        s2_vg, c_vg, s2_cg, c_cg, h1_g, Wr2_vg, b2_g, Wr2_cg)
    o_p = _combine_call(npp, _O, 1, 0, False, False, False, _O)(
        s2_vp, c_vp, h1_p, Wr2_vp, bl2_vp[None])
    o_c = _combine_call(ncp, _O, 1, 0, False, False, False, _O)(
        s2_gc, c_gc, h1_c, Wr2_gc, bl2_gc[None])
    b2_v = (bl2_gv + bl2_pv)[None]
    o_v = _combine_call(nvp, _O, 2, 2, False, False, False, _O)(
        *s2_gv, c_gv, *s2_pv, c_pv, h1_v, Wr2_gv, b2_v, Wr2_pv)

    return (o_v[:_NV], o_g[:_NG], o_c[:_NC], o_p[:_NP])
